# baseline (speedup 1.0000x reference)
import functools

import jax
import jax.numpy as jnp
from jax import lax
from jax.experimental import pallas as pl
from jax.experimental.pallas import tpu as pltpu
from jax.experimental.pallas import tpu_sc as plsc

T = 8192
D = 2048
E = 16
K = 2
TM = 512
CH = 1
CHUNK = T // CH
NC = 2
NS = 16
NW = NC * NS
TPW = CHUNK // NW
L = 16
G = TPW // L


def _matmul_body(x_ref, w_ref, o_ref):
    lg = lax.dot_general(
        x_ref[...], w_ref[...], (((1,), (1,)), ((), ())),
        preferred_element_type=jnp.float32)
    o_ref[...] = jnp.concatenate(
        [lg[j * (TM // 8):(j + 1) * (TM // 8), :] for j in range(8)], axis=1)


def _logits(x, W, c):
    off = c * (CHUNK // TM)
    return pl.pallas_call(
        _matmul_body,
        grid=(CHUNK // TM,),
        in_specs=[
            pl.BlockSpec((TM, D), lambda i: (i + off, 0)),
            pl.BlockSpec((E, D), lambda i: (0, 0)),
        ],
        out_specs=pl.BlockSpec((TM // 8, 128), lambda i: (i, 0)),
        out_shape=jax.ShapeDtypeStruct((CHUNK // 8, 128), jnp.float32),
    )(x, W)


_mesh = plsc.VectorSubcoreMesh(
    core_axis_name="c", subcore_axis_name="s", num_cores=NC, num_subcores=NS)


@functools.partial(
    pl.kernel,
    out_type=(jax.ShapeDtypeStruct((CHUNK * K // 128, 128), jnp.float32),
              jax.ShapeDtypeStruct((CHUNK * K // 128, 128), jnp.int32)),
    mesh=_mesh,
    scratch_types=[
        pltpu.VMEM((TM * E,), jnp.float32),
        pltpu.VMEM((TPW * K // 128, 128), jnp.float32),
        pltpu.VMEM((TPW * K // 128, 128), jnp.int32),
    ],
    compiler_params=pltpu.CompilerParams(needs_layout_passes=False),
)
def _router(logits_hbm, w_hbm, i_hbm, lg_v, w_v, i_v):
    wid = lax.axis_index("s") * NC + lax.axis_index("c")
    base = wid * TPW
    h = wid % (TM // TPW) if TM > TPW else 0
    blk = base // TM
    pltpu.sync_copy(logits_hbm.at[pl.ds(blk * TM * E, TM * E)], lg_v)

    def group(g, carry):
        tok = jnp.full((L,), g * L, jnp.int32) + lax.iota(jnp.int32, L)
        gb = (lax.rem(g, 4) * 2048 + 64 * h + lax.div(g, 4) * 16)
        abase = jnp.full((L,), gb, jnp.int32) + lax.iota(jnp.int32, L) * jnp.full((L,), 128, jnp.int32)
        ls = [plsc.load_gather(lg_v, [abase + jnp.full((L,), e, jnp.int32)])
              for e in range(E)]
        m1 = ls[0]
        i1 = jnp.zeros((L,), jnp.int32)
        m2 = jnp.full((L,), -jnp.inf, jnp.float32)
        i2 = jnp.zeros((L,), jnp.int32)
        for e in range(1, E):
            v = ls[e]
            ev = jnp.full((L,), e, jnp.int32)
            gt1 = v > m1
            gt2 = v > m2
            m2 = jnp.where(gt1, m1, jnp.where(gt2, v, m2))
            i2 = jnp.where(gt1, i1, jnp.where(gt2, ev, i2))
            m1 = jnp.where(gt1, v, m1)
            i1 = jnp.where(gt1, ev, i1)
        z = jnp.full((L,), 0.0, jnp.float32)
        for e in range(E):
            z = z + jnp.exp(ls[e] - m1)
        e2 = jnp.exp(m2 - m1)
        one = jnp.full((L,), 1.0, jnp.float32)
        denom = one + e2 + jnp.full((L,), 1e-9, jnp.float32) * z
        w1 = one / denom
        w2 = e2 / denom
        orow = jnp.full((L,), lax.div(g, 4), jnp.int32)
        ocol = (jnp.full((L,), lax.rem(g, 4) * 32, jnp.int32)
                + lax.iota(jnp.int32, L) * jnp.full((L,), 2, jnp.int32))
        one_i = jnp.full((L,), 1, jnp.int32)
        plsc.store_scatter(w_v, [orow, ocol], w1)
        plsc.store_scatter(w_v, [orow, ocol + one_i], w2)
        plsc.store_scatter(i_v, [orow, ocol], i1)
        plsc.store_scatter(i_v, [orow, ocol + one_i], i2)
        return carry

    lax.fori_loop(0, G, group, 0)
    orows = TPW * K // 128
    pltpu.sync_copy(w_v, w_hbm.at[pl.ds(wid * orows, orows)])
    pltpu.sync_copy(i_v, i_hbm.at[pl.ds(wid * orows, orows)])


def kernel(x, W):
    ws, idxs = [], []
    for c in range(CH):
        lg = _logits(x, W, c)
        wc, ic = _router(lg.reshape(CHUNK * E))
        ws.append(wc.reshape(CHUNK, K))
        idxs.append(ic.reshape(CHUNK, K))
    if CH == 1:
        return (ws[0], idxs[0])
    return (jnp.concatenate(ws, axis=0), jnp.concatenate(idxs, axis=0))


# speedup vs baseline: 1.0850x; 1.0850x over previous
import functools

import jax
import jax.numpy as jnp
from jax import lax
from jax.experimental import pallas as pl
from jax.experimental.pallas import tpu as pltpu
from jax.experimental.pallas import tpu_sc as plsc

T = 8192
D = 2048
E = 16
K = 2
TM = 512
CH = 1
CHUNK = T // CH
NC = 2
NS = 16
NW = NC * NS
TPW = CHUNK // NW
L = 16
G = TPW // L


def _matmul_body(x_ref, w_ref, o_ref):
    lg = lax.dot_general(
        x_ref[...], w_ref[...], (((1,), (1,)), ((), ())),
        preferred_element_type=jnp.float32)
    o_ref[...] = jnp.concatenate(
        [lg[j * (TM // 8):(j + 1) * (TM // 8), :] for j in range(8)], axis=1)


def _logits(x, W, c):
    off = c * (CHUNK // TM)
    return pl.pallas_call(
        _matmul_body,
        grid=(CHUNK // TM,),
        in_specs=[
            pl.BlockSpec((TM, D), lambda i: (i + off, 0)),
            pl.BlockSpec((E, D), lambda i: (0, 0)),
        ],
        out_specs=pl.BlockSpec((TM // 8, 128), lambda i: (i, 0)),
        out_shape=jax.ShapeDtypeStruct((CHUNK // 8, 128), jnp.float32),
    )(x, W)


_mesh = plsc.VectorSubcoreMesh(
    core_axis_name="c", subcore_axis_name="s", num_cores=NC, num_subcores=NS)


@functools.partial(
    pl.kernel,
    out_type=(jax.ShapeDtypeStruct((CHUNK, K), jnp.float32),
              jax.ShapeDtypeStruct((CHUNK, K), jnp.int32)),
    mesh=_mesh,
    scratch_types=[
        pltpu.VMEM((TM * E,), jnp.float32),
        pltpu.VMEM((TPW, K), jnp.float32),
        pltpu.VMEM((TPW, K), jnp.int32),
    ],
    compiler_params=pltpu.CompilerParams(needs_layout_passes=False),
)
def _router(logits_hbm, w_hbm, i_hbm, lg_v, w_v, i_v):
    wid = lax.axis_index("s") * NC + lax.axis_index("c")
    base = wid * TPW
    h = wid % (TM // TPW) if TM > TPW else 0
    blk = base // TM
    pltpu.sync_copy(logits_hbm.at[pl.ds(blk * TM * E, TM * E)], lg_v)

    def group(g, carry):
        tok = jnp.full((L,), g * L, jnp.int32) + lax.iota(jnp.int32, L)
        gb = (lax.rem(g, 4) * 2048 + 64 * h + lax.div(g, 4) * 16)
        abase = jnp.full((L,), gb, jnp.int32) + lax.iota(jnp.int32, L) * jnp.full((L,), 128, jnp.int32)
        ls = [plsc.load_gather(lg_v, [abase + jnp.full((L,), e, jnp.int32)])
              for e in range(E)]
        m1 = ls[0]
        i1 = jnp.zeros((L,), jnp.int32)
        m2 = jnp.full((L,), -jnp.inf, jnp.float32)
        i2 = jnp.zeros((L,), jnp.int32)
        for e in range(1, E):
            v = ls[e]
            ev = jnp.full((L,), e, jnp.int32)
            gt1 = v > m1
            gt2 = v > m2
            m2 = jnp.where(gt1, m1, jnp.where(gt2, v, m2))
            i2 = jnp.where(gt1, i1, jnp.where(gt2, ev, i2))
            m1 = jnp.where(gt1, v, m1)
            i1 = jnp.where(gt1, ev, i1)
        e2 = jnp.exp(m2 - m1)
        one = jnp.full((L,), 1.0, jnp.float32)
        denom = one + e2
        w1 = one / denom
        w2 = e2 / denom
        col0 = jnp.zeros((L,), jnp.int32)
        col1 = jnp.full((L,), 1, jnp.int32)
        plsc.store_scatter(w_v, [tok, col0], w1)
        plsc.store_scatter(w_v, [tok, col1], w2)
        plsc.store_scatter(i_v, [tok, col0], i1)
        plsc.store_scatter(i_v, [tok, col1], i2)
        return carry

    lax.fori_loop(0, G, group, 0)
    pltpu.sync_copy(w_v, w_hbm.at[pl.ds(base, TPW)])
    pltpu.sync_copy(i_v, i_hbm.at[pl.ds(base, TPW)])


def kernel(x, W):
    ws, idxs = [], []
    for c in range(CH):
        lg = _logits(x, W, c)
        wc, ic = _router(lg.reshape(CHUNK * E))
        ws.append(wc)
        idxs.append(ic)
    if CH == 1:
        return (ws[0], idxs[0])
    return (jnp.concatenate(ws, axis=0), jnp.concatenate(idxs, axis=0))


# speedup vs baseline: 1.1506x; 1.0604x over previous
import functools

import jax
import jax.numpy as jnp
from jax import lax
from jax.experimental import pallas as pl
from jax.experimental.pallas import tpu as pltpu
from jax.experimental.pallas import tpu_sc as plsc

T = 8192
D = 2048
E = 16
K = 2
TM = 1024
CH = 1
CHUNK = T // CH
NC = 2
NS = 16
NW = NC * NS
TPW = CHUNK // NW
L = 16
G = TPW // L


def _matmul_body(x_ref, w_ref, o_ref):
    lg = lax.dot_general(
        x_ref[...], w_ref[...], (((1,), (1,)), ((), ())),
        preferred_element_type=jnp.float32)
    o_ref[...] = jnp.concatenate(
        [lg[j * (TM // 8):(j + 1) * (TM // 8), :] for j in range(8)], axis=1)


def _logits(x, W, c):
    off = c * (CHUNK // TM)
    return pl.pallas_call(
        _matmul_body,
        grid=(CHUNK // TM,),
        in_specs=[
            pl.BlockSpec((TM, D), lambda i: (i + off, 0)),
            pl.BlockSpec((E, D), lambda i: (0, 0)),
        ],
        out_specs=pl.BlockSpec((TM // 8, 128), lambda i: (i, 0)),
        out_shape=jax.ShapeDtypeStruct((CHUNK // 8, 128), jnp.float32),
    )(x, W)


_mesh = plsc.VectorSubcoreMesh(
    core_axis_name="c", subcore_axis_name="s", num_cores=NC, num_subcores=NS)


@functools.partial(
    pl.kernel,
    out_type=(jax.ShapeDtypeStruct((CHUNK, K), jnp.float32),
              jax.ShapeDtypeStruct((CHUNK, K), jnp.int32)),
    mesh=_mesh,
    scratch_types=[
        pltpu.VMEM((TM * E,), jnp.float32),
        pltpu.VMEM((TPW, K), jnp.float32),
        pltpu.VMEM((TPW, K), jnp.int32),
    ],
    compiler_params=pltpu.CompilerParams(needs_layout_passes=False),
)
def _router(logits_hbm, w_hbm, i_hbm, lg_v, w_v, i_v):
    wid = lax.axis_index("s") * NC + lax.axis_index("c")
    base = wid * TPW
    h = wid % (TM // TPW) if TM > TPW else 0
    blk = base // TM
    pltpu.sync_copy(logits_hbm.at[pl.ds(blk * TM * E, TM * E)], lg_v)

    def group(g, carry):
        tok = jnp.full((L,), g * L, jnp.int32) + lax.iota(jnp.int32, L)
        gb = (lax.rem(g, 4) * 2048 + 64 * h + lax.div(g, 4) * 16)
        abase = jnp.full((L,), gb, jnp.int32) + lax.iota(jnp.int32, L) * jnp.full((L,), 128, jnp.int32)
        ls = [plsc.load_gather(lg_v, [abase + jnp.full((L,), e, jnp.int32)])
              for e in range(E)]
        m1 = ls[0]
        i1 = jnp.zeros((L,), jnp.int32)
        m2 = jnp.full((L,), -jnp.inf, jnp.float32)
        i2 = jnp.zeros((L,), jnp.int32)
        for e in range(1, E):
            v = ls[e]
            ev = jnp.full((L,), e, jnp.int32)
            gt1 = v > m1
            gt2 = v > m2
            m2 = jnp.where(gt1, m1, jnp.where(gt2, v, m2))
            i2 = jnp.where(gt1, i1, jnp.where(gt2, ev, i2))
            m1 = jnp.where(gt1, v, m1)
            i1 = jnp.where(gt1, ev, i1)
        e2 = jnp.exp(m2 - m1)
        one = jnp.full((L,), 1.0, jnp.float32)
        denom = one + e2
        w1 = one / denom
        w2 = e2 / denom
        col0 = jnp.zeros((L,), jnp.int32)
        col1 = jnp.full((L,), 1, jnp.int32)
        plsc.store_scatter(w_v, [tok, col0], w1)
        plsc.store_scatter(w_v, [tok, col1], w2)
        plsc.store_scatter(i_v, [tok, col0], i1)
        plsc.store_scatter(i_v, [tok, col1], i2)
        return carry

    lax.fori_loop(0, G, group, 0)
    pltpu.sync_copy(w_v, w_hbm.at[pl.ds(base, TPW)])
    pltpu.sync_copy(i_v, i_hbm.at[pl.ds(base, TPW)])


def kernel(x, W):
    ws, idxs = [], []
    for c in range(CH):
        lg = _logits(x, W, c)
        wc, ic = _router(lg.reshape(CHUNK * E))
        ws.append(wc)
        idxs.append(ic)
    if CH == 1:
        return (ws[0], idxs[0])
    return (jnp.concatenate(ws, axis=0), jnp.concatenate(idxs, axis=0))


# speedup vs baseline: 1.1540x; 1.0030x over previous
import functools

import jax
import jax.numpy as jnp
from jax import lax
from jax.experimental import pallas as pl
from jax.experimental.pallas import tpu as pltpu
from jax.experimental.pallas import tpu_sc as plsc

T = 8192
D = 2048
E = 16
K = 2
TM = 1024
CH = 1
CHUNK = T // CH
NC = 2
NS = 16
NW = NC * NS
TPW = CHUNK // NW
L = 16
G = TPW // L
S = TM // 8
GP = S // L


def _matmul_body(x_ref, w_ref, o_ref):
    lg = lax.dot_general(
        x_ref[...], w_ref[...], (((1,), (1,)), ((), ())),
        preferred_element_type=jnp.float32)
    o_ref[...] = jnp.concatenate(
        [lg[j * (TM // 8):(j + 1) * (TM // 8), :] for j in range(8)], axis=1)


def _logits(x, W, c):
    off = c * (CHUNK // TM)
    return pl.pallas_call(
        _matmul_body,
        grid=(CHUNK // TM,),
        in_specs=[
            pl.BlockSpec((TM, D), lambda i: (i + off, 0)),
            pl.BlockSpec((E, D), lambda i: (0, 0)),
        ],
        out_specs=pl.BlockSpec((TM // 8, 128), lambda i: (i, 0)),
        out_shape=jax.ShapeDtypeStruct((CHUNK // 8, 128), jnp.float32),
    )(x, W)


_mesh = plsc.VectorSubcoreMesh(
    core_axis_name="c", subcore_axis_name="s", num_cores=NC, num_subcores=NS)


@functools.partial(
    pl.kernel,
    out_type=(jax.ShapeDtypeStruct((CHUNK, K), jnp.float32),
              jax.ShapeDtypeStruct((CHUNK, K), jnp.int32)),
    mesh=_mesh,
    scratch_types=[
        pltpu.VMEM((TM * E,), jnp.float32),
        pltpu.VMEM((TPW, K), jnp.float32),
        pltpu.VMEM((TPW, K), jnp.int32),
    ],
    compiler_params=pltpu.CompilerParams(needs_layout_passes=False),
)
def _router(logits_hbm, w_hbm, i_hbm, lg_v, w_v, i_v):
    wid = lax.axis_index("s") * NC + lax.axis_index("c")
    base = wid * TPW
    h = wid % (TM // TPW)
    blk = wid // (TM // TPW)
    pltpu.sync_copy(logits_hbm.at[pl.ds(blk * TM * E, TM * E)], lg_v)

    def group(g, carry):
        tok = jnp.full((L,), g * L, jnp.int32) + lax.iota(jnp.int32, L)
        gb = (lax.rem(g, GP) * 2048
              + (16 * (TPW // S)) * h + lax.div(g, GP) * 16)
        abase = jnp.full((L,), gb, jnp.int32) + lax.iota(jnp.int32, L) * jnp.full((L,), 128, jnp.int32)
        ls = [plsc.load_gather(lg_v, [abase + jnp.full((L,), e, jnp.int32)])
              for e in range(E)]
        m1 = ls[0]
        i1 = jnp.zeros((L,), jnp.int32)
        m2 = jnp.full((L,), -jnp.inf, jnp.float32)
        i2 = jnp.zeros((L,), jnp.int32)
        for e in range(1, E):
            v = ls[e]
            ev = jnp.full((L,), e, jnp.int32)
            gt1 = v > m1
            gt2 = v > m2
            m2 = jnp.where(gt1, m1, jnp.where(gt2, v, m2))
            i2 = jnp.where(gt1, i1, jnp.where(gt2, ev, i2))
            m1 = jnp.where(gt1, v, m1)
            i1 = jnp.where(gt1, ev, i1)
        e2 = jnp.exp(m2 - m1)
        one = jnp.full((L,), 1.0, jnp.float32)
        denom = one + e2
        w1 = one / denom
        w2 = e2 / denom
        col0 = jnp.zeros((L,), jnp.int32)
        col1 = jnp.full((L,), 1, jnp.int32)
        plsc.store_scatter(w_v, [tok, col0], w1)
        plsc.store_scatter(w_v, [tok, col1], w2)
        plsc.store_scatter(i_v, [tok, col0], i1)
        plsc.store_scatter(i_v, [tok, col1], i2)
        return carry

    lax.fori_loop(0, G, group, 0)
    pltpu.sync_copy(w_v, w_hbm.at[pl.ds(base, TPW)])
    pltpu.sync_copy(i_v, i_hbm.at[pl.ds(base, TPW)])


def kernel(x, W):
    ws, idxs = [], []
    for c in range(CH):
        lg = _logits(x, W, c)
        wc, ic = _router(lg.reshape(CHUNK * E))
        ws.append(wc)
        idxs.append(ic)
    if CH == 1:
        return (ws[0], idxs[0])
    return (jnp.concatenate(ws, axis=0), jnp.concatenate(idxs, axis=0))
